# baseline (speedup 1.0000x reference)
import functools

import jax
import jax.numpy as jnp
from jax.experimental import pallas as pl


def _silu(v):
    s = v * 0.5
    return s * (jnp.tanh(s) + 1.0)


def _layer_body(h_ref, hi_ref, xi_ref, xTa_ref, xTb_ref, xjf_ref,
                we1a_ref, we1b_ref, we1ca_ref, we1cb_ref, be1_ref,
                we2_ref, we2d_ref, be2_ref, be2d_ref,
                wx1_ref, wx1d_ref, bx1_ref, bx1d_ref,
                wx2r_ref, wx2w_ref, bx2_ref, xsum_ref,
                wh1a_ref, wh1b_ref, bh1_ref, wh2_ref, bh2_ref,
                oh_ref, ox_ref, *, bi, n, d):
    f32 = jnp.float32
    nh = n // 2

    h_all = h_ref[:, :]
    hi = hi_ref[:, :]
    xi = xi_ref[:, :]

    d2a = jnp.zeros((bi, nh), f32)
    d2b = jnp.zeros((bi, nh), f32)
    for k in range(3):
        dka = xi[:, k:k + 1] - xTa_ref[k:k + 1, :]
        dkb = xi[:, k:k + 1] - xTb_ref[k:k + 1, :]
        d2a = d2a + dka * dka
        d2b = d2b + dkb * dkb

    ai = jnp.dot(hi, we1a_ref[:, :], preferred_element_type=f32)
    bj = jnp.dot(h_all, we1b_ref[:, :], preferred_element_type=f32)
    aip = ai + be1_ref[0, :][None, :]
    aip2 = jnp.concatenate([aip, aip], axis=1)
    bjp2 = jnp.concatenate([bj[:nh, :], bj[nh:, :]], axis=1)
    m0 = (aip2[:, None, :] + bjp2[None, :, :]
          + d2a[:, :, None] * we1ca_ref[0, :][None, None, :]
          + d2b[:, :, None] * we1cb_ref[0, :][None, None, :])
    m1 = _silu(m0).reshape(bi * nh, 2 * d)
    m = _silu(jnp.dot(m1, we2d_ref[:, :], preferred_element_type=f32)
              + be2d_ref[0, :][None, :])

    t = _silu(jnp.dot(m, wx1d_ref[:, :], preferred_element_type=f32)
              + bx1d_ref[0, :][None, :])
    cc = jnp.dot(t, wx2w_ref[:, :], preferred_element_type=f32)
    u = cc * xjf_ref[:, :]
    cvw = jnp.sum(u.reshape(bi, nh, 2 * d), axis=1)
    cv = cvw[:, 0:4] + cvw[:, 4:8]
    bx2 = bx2_ref[0, 0]
    cxu = cv[:, 0:3] + bx2 * xsum_ref[0, :][None, :]
    csumu = cv[:, 3:4] + n * bx2

    maggp = jnp.sum(m.reshape(bi, nh, 2 * d), axis=1)
    maggu = maggp[:, :d] + maggp[:, d:]

    bj_diag = jnp.dot(hi, we1b_ref[:, :], preferred_element_type=f32)
    m0d = aip + bj_diag
    md = _silu(jnp.dot(_silu(m0d), we2_ref[:, :], preferred_element_type=f32)
               + be2_ref[0, :][None, :])
    td = _silu(jnp.dot(md, wx1_ref[:, :], preferred_element_type=f32)
               + bx1_ref[0, :][None, :])
    cd = jnp.sum(td * wx2r_ref[0, :][None, :], axis=1, keepdims=True) + bx2

    csum = csumu - cd
    cx = cxu - cd * xi
    ox_ref[:, :] = xi + (xi * csum - cx) * (1.0 / (n - 1))

    magg = maggu - md
    g = _silu(jnp.dot(hi, wh1a_ref[:, :], preferred_element_type=f32)
              + jnp.dot(magg, wh1b_ref[:, :], preferred_element_type=f32)
              + bh1_ref[0, :][None, :])
    hupd = jnp.dot(g, wh2_ref[:, :], preferred_element_type=f32) \
        + bh2_ref[0, :][None, :]
    oh_ref[:, :] = _silu(hi + hupd)


def _egnn_layer(h, x, We1, be1, We2, be2, Wx1, bx1, Wx2, bx2,
                Wh1, bh1, Wh2, bh2, *, bi, interpret=False):
    n, d = h.shape
    f32 = jnp.float32
    nh = n // 2
    xTa = x[:nh].T
    xTb = x[nh:].T
    one = jnp.ones((nh, 1), f32)
    xjf = jnp.tile(jnp.concatenate([x[:nh], one, x[nh:], one], axis=1),
                   (bi, 2 * d // 8))
    xsum = jnp.sum(x, axis=0, keepdims=True)

    we1a = We1[:d]
    we1b = We1[d:2 * d]
    we1c = We1[2 * d:2 * d + 1]
    z1 = jnp.zeros((1, d), f32)
    we1ca = jnp.concatenate([we1c, z1], axis=1)
    we1cb = jnp.concatenate([z1, we1c], axis=1)
    zd = jnp.zeros((d, d), f32)
    we2d = jnp.concatenate(
        [jnp.concatenate([We2, zd], axis=1),
         jnp.concatenate([zd, We2], axis=1)], axis=0)
    wx1d = jnp.concatenate(
        [jnp.concatenate([Wx1, zd], axis=1),
         jnp.concatenate([zd, Wx1], axis=1)], axis=0)
    be2d = jnp.tile(be2.reshape(1, d), (1, 2))
    bx1d = jnp.tile(bx1.reshape(1, d), (1, 2))
    zc = jnp.zeros((d, 1), f32)
    colA = jnp.concatenate([Wx2, zc], axis=0)
    colB = jnp.concatenate([zc, Wx2], axis=0)
    wx2w = jnp.tile(jnp.concatenate([colA] * 4 + [colB] * 4, axis=1),
                    (1, 2 * d // 8))
    wx2r = Wx2.T
    bx2m = bx2.reshape(1, 1)
    wh1a = Wh1[:d]
    wh1b = Wh1[d:]

    full = lambda shape: pl.BlockSpec(shape, lambda i: (0, 0))
    body = functools.partial(_layer_body, bi=bi, n=n, d=d)
    return pl.pallas_call(
        body,
        grid=(n // bi,),
        in_specs=[
            full((n, d)),
            pl.BlockSpec((bi, d), lambda i: (i, 0)),
            pl.BlockSpec((bi, 3), lambda i: (i, 0)),
            full((3, nh)),
            full((3, nh)),
            full((bi * nh, 2 * d)),
            full((d, d)),
            full((d, d)),
            full((1, 2 * d)),
            full((1, 2 * d)),
            full((1, d)),
            full((d, d)),
            full((2 * d, 2 * d)),
            full((1, d)),
            full((1, 2 * d)),
            full((d, d)),
            full((2 * d, 2 * d)),
            full((1, d)),
            full((1, 2 * d)),
            full((1, d)),
            full((2 * d, 2 * d)),
            full((1, 1)),
            full((1, 3)),
            full((d, d)),
            full((d, d)),
            full((1, d)),
            full((d, d)),
            full((1, d)),
        ],
        out_specs=[
            pl.BlockSpec((bi, d), lambda i: (i, 0)),
            pl.BlockSpec((bi, 3), lambda i: (i, 0)),
        ],
        out_shape=[
            jax.ShapeDtypeStruct((n, d), jnp.float32),
            jax.ShapeDtypeStruct((n, 3), jnp.float32),
        ],
        interpret=interpret,
    )(h, h, x, xTa, xTb, xjf, we1a, we1b, we1ca, we1cb, be1.reshape(1, d),
      We2, we2d, be2.reshape(1, d), be2d, Wx1, wx1d, bx1.reshape(1, d), bx1d,
      wx2r, wx2w, bx2m, xsum, wh1a, wh1b, bh1.reshape(1, d),
      Wh2, bh2.reshape(1, d))


def kernel(h, x, We1, be1, We2, be2, Wx1, bx1, Wx2, bx2, Wh1, bh1, Wh2, bh2):
    L = We1.shape[0]
    for l in range(L):
        h, x = _egnn_layer(h, x, We1[l], be1[l], We2[l], be2[l],
                           Wx1[l], bx1[l], Wx2[l], bx2[l],
                           Wh1[l], bh1[l], Wh2[l], bh2[l], bi=64)
    return (h, x)


# speedup vs baseline: 1.0730x; 1.0730x over previous
import functools

import jax
import jax.numpy as jnp
from jax.experimental import pallas as pl


def _silu(v):
    s = v * 0.5
    return s * (jnp.tanh(s) + 1.0)


def _layer_body(h_ref, hi_ref, xi_ref, xTa_ref, xTb_ref, xjf_ref,
                we1a_ref, we1b_ref, we1ca_ref, we1cb_ref, be1_ref,
                we2_ref, we2d_ref, be2_ref, be2d_ref,
                wx1_ref, wx1d_ref, bx1_ref, bx1d_ref,
                wx2r_ref, wx2w_ref, bx2_ref, xsum_ref,
                wh1a_ref, wh1b_ref, bh1_ref, wh2_ref, bh2_ref,
                oh_ref, ox_ref, *, bi, n, d):
    f32 = jnp.float32
    nh = n // 2

    h_all = h_ref[:, :]
    hi = hi_ref[:, :]
    xi = xi_ref[:, :]

    d2a = jnp.zeros((bi, nh), f32)
    d2b = jnp.zeros((bi, nh), f32)
    for k in range(3):
        dka = xi[:, k:k + 1] - xTa_ref[k:k + 1, :]
        dkb = xi[:, k:k + 1] - xTb_ref[k:k + 1, :]
        d2a = d2a + dka * dka
        d2b = d2b + dkb * dkb

    ai = jnp.dot(hi, we1a_ref[:, :], preferred_element_type=f32)
    bj = jnp.dot(h_all, we1b_ref[:, :], preferred_element_type=f32)
    aip = ai + be1_ref[0, :][None, :]
    aip2 = jnp.concatenate([aip, aip], axis=1)
    bjp2 = jnp.concatenate([bj[:nh, :], bj[nh:, :]], axis=1)
    m0 = (aip2[:, None, :] + bjp2[None, :, :]
          + d2a[:, :, None] * we1ca_ref[0, :][None, None, :]
          + d2b[:, :, None] * we1cb_ref[0, :][None, None, :])
    m1 = _silu(m0).reshape(bi * nh, 2 * d)
    m = _silu(jnp.dot(m1, we2d_ref[:, :], preferred_element_type=f32)
              + be2d_ref[0, :][None, :])

    t = _silu(jnp.dot(m, wx1d_ref[:, :], preferred_element_type=f32)
              + bx1d_ref[0, :][None, :])
    cc = jnp.dot(t, wx2w_ref[:, :], preferred_element_type=f32)
    u = cc * xjf_ref[:, :]
    cvw = jnp.sum(u.reshape(bi, nh, 2 * d), axis=1)
    cv = cvw[:, 0:4] + cvw[:, 4:8]
    bx2 = bx2_ref[0, 0]
    cxu = cv[:, 0:3] + bx2 * xsum_ref[0, :][None, :]
    csumu = cv[:, 3:4] + n * bx2

    maggp = jnp.sum(m.reshape(bi, nh, 2 * d), axis=1)
    maggu = maggp[:, :d] + maggp[:, d:]

    bj_diag = jnp.dot(hi, we1b_ref[:, :], preferred_element_type=f32)
    m0d = aip + bj_diag
    md = _silu(jnp.dot(_silu(m0d), we2_ref[:, :], preferred_element_type=f32)
               + be2_ref[0, :][None, :])
    td = _silu(jnp.dot(md, wx1_ref[:, :], preferred_element_type=f32)
               + bx1_ref[0, :][None, :])
    cd = jnp.sum(td * wx2r_ref[0, :][None, :], axis=1, keepdims=True) + bx2

    csum = csumu - cd
    cx = cxu - cd * xi
    ox_ref[:, :] = xi + (xi * csum - cx) * (1.0 / (n - 1))

    magg = maggu - md
    g = _silu(jnp.dot(hi, wh1a_ref[:, :], preferred_element_type=f32)
              + jnp.dot(magg, wh1b_ref[:, :], preferred_element_type=f32)
              + bh1_ref[0, :][None, :])
    hupd = jnp.dot(g, wh2_ref[:, :], preferred_element_type=f32) \
        + bh2_ref[0, :][None, :]
    oh_ref[:, :] = _silu(hi + hupd)


def _egnn_layer(h, x, We1, be1, We2, be2, Wx1, bx1, Wx2, bx2,
                Wh1, bh1, Wh2, bh2, *, bi, interpret=False):
    n, d = h.shape
    f32 = jnp.float32
    nh = n // 2
    xTa = x[:nh].T
    xTb = x[nh:].T
    one = jnp.ones((nh, 1), f32)
    xjf = jnp.tile(jnp.concatenate([x[:nh], one, x[nh:], one], axis=1),
                   (bi, 2 * d // 8))
    xsum = jnp.sum(x, axis=0, keepdims=True)

    we1a = We1[:d]
    we1b = We1[d:2 * d]
    we1c = We1[2 * d:2 * d + 1]
    z1 = jnp.zeros((1, d), f32)
    we1ca = jnp.concatenate([we1c, z1], axis=1)
    we1cb = jnp.concatenate([z1, we1c], axis=1)
    zd = jnp.zeros((d, d), f32)
    we2d = jnp.concatenate(
        [jnp.concatenate([We2, zd], axis=1),
         jnp.concatenate([zd, We2], axis=1)], axis=0)
    wx1d = jnp.concatenate(
        [jnp.concatenate([Wx1, zd], axis=1),
         jnp.concatenate([zd, Wx1], axis=1)], axis=0)
    be2d = jnp.tile(be2.reshape(1, d), (1, 2))
    bx1d = jnp.tile(bx1.reshape(1, d), (1, 2))
    zc = jnp.zeros((d, 1), f32)
    colA = jnp.concatenate([Wx2, zc], axis=0)
    colB = jnp.concatenate([zc, Wx2], axis=0)
    wx2w = jnp.tile(jnp.concatenate([colA] * 4 + [colB] * 4, axis=1),
                    (1, 2 * d // 8))
    wx2r = Wx2.T
    bx2m = bx2.reshape(1, 1)
    wh1a = Wh1[:d]
    wh1b = Wh1[d:]

    full = lambda shape: pl.BlockSpec(shape, lambda i: (0, 0))
    body = functools.partial(_layer_body, bi=bi, n=n, d=d)
    return pl.pallas_call(
        body,
        grid=(n // bi,),
        in_specs=[
            full((n, d)),
            pl.BlockSpec((bi, d), lambda i: (i, 0)),
            pl.BlockSpec((bi, 3), lambda i: (i, 0)),
            full((3, nh)),
            full((3, nh)),
            full((bi * nh, 2 * d)),
            full((d, d)),
            full((d, d)),
            full((1, 2 * d)),
            full((1, 2 * d)),
            full((1, d)),
            full((d, d)),
            full((2 * d, 2 * d)),
            full((1, d)),
            full((1, 2 * d)),
            full((d, d)),
            full((2 * d, 2 * d)),
            full((1, d)),
            full((1, 2 * d)),
            full((1, d)),
            full((2 * d, 2 * d)),
            full((1, 1)),
            full((1, 3)),
            full((d, d)),
            full((d, d)),
            full((1, d)),
            full((d, d)),
            full((1, d)),
        ],
        out_specs=[
            pl.BlockSpec((bi, d), lambda i: (i, 0)),
            pl.BlockSpec((bi, 3), lambda i: (i, 0)),
        ],
        out_shape=[
            jax.ShapeDtypeStruct((n, d), jnp.float32),
            jax.ShapeDtypeStruct((n, 3), jnp.float32),
        ],
        interpret=interpret,
    )(h, h, x, xTa, xTb, xjf, we1a, we1b, we1ca, we1cb, be1.reshape(1, d),
      We2, we2d, be2.reshape(1, d), be2d, Wx1, wx1d, bx1.reshape(1, d), bx1d,
      wx2r, wx2w, bx2m, xsum, wh1a, wh1b, bh1.reshape(1, d),
      Wh2, bh2.reshape(1, d))


def kernel(h, x, We1, be1, We2, be2, Wx1, bx1, Wx2, bx2, Wh1, bh1, Wh2, bh2):
    L = We1.shape[0]
    for l in range(L):
        h, x = _egnn_layer(h, x, We1[l], be1[l], We2[l], be2[l],
                           Wx1[l], bx1[l], Wx2[l], bx2[l],
                           Wh1[l], bh1[l], Wh2[l], bh2[l], bi=16)
    return (h, x)


# speedup vs baseline: 1.1323x; 1.0553x over previous
import functools

import jax
import jax.numpy as jnp
from jax.experimental import pallas as pl


def _silu(v):
    s = v * 0.5
    return s * jnp.tanh(s) + s


def _layer_body(h_ref, hi_ref, xi_ref, xTa_ref, xTb_ref, xjf_ref,
                we1a_ref, we1b_ref, we1ca_ref, we1cb_ref, be1_ref,
                we2_ref, we2d_ref, be2_ref, be2d_ref,
                wx1_ref, wx1d_ref, bx1_ref, bx1d_ref,
                wx2r_ref, wx2w_ref, bx2_ref, xsum_ref,
                wh1a_ref, wh1b_ref, bh1_ref, wh2_ref, bh2_ref,
                oh_ref, ox_ref, *, bi, n, d):
    f32 = jnp.float32
    nh = n // 2

    h_all = h_ref[:, :]
    hi = hi_ref[:, :]
    xi = xi_ref[:, :]

    d2a = jnp.zeros((bi, nh), f32)
    d2b = jnp.zeros((bi, nh), f32)
    for k in range(3):
        dka = xi[:, k:k + 1] - xTa_ref[k:k + 1, :]
        dkb = xi[:, k:k + 1] - xTb_ref[k:k + 1, :]
        d2a = d2a + dka * dka
        d2b = d2b + dkb * dkb

    ai = jnp.dot(hi, we1a_ref[:, :], preferred_element_type=f32)
    bj = jnp.dot(h_all, we1b_ref[:, :], preferred_element_type=f32)
    aip = ai + be1_ref[0, :][None, :]
    aip2 = jnp.concatenate([aip, aip], axis=1)
    bjp2 = jnp.concatenate([bj[:nh, :], bj[nh:, :]], axis=1)
    m0 = (aip2[:, None, :] + bjp2[None, :, :]
          + d2a[:, :, None] * we1ca_ref[0, :][None, None, :]
          + d2b[:, :, None] * we1cb_ref[0, :][None, None, :])
    m1 = _silu(m0).reshape(bi * nh, 2 * d)
    m = _silu(jnp.dot(m1, we2d_ref[:, :], preferred_element_type=f32)
              + be2d_ref[0, :][None, :])

    t = _silu(jnp.dot(m, wx1d_ref[:, :], preferred_element_type=f32)
              + bx1d_ref[0, :][None, :])
    cc = jnp.dot(t, wx2w_ref[:, :], preferred_element_type=f32)
    u = cc * xjf_ref[:, :]
    cvw = jnp.sum(u.reshape(bi, nh, 2 * d), axis=1)
    cv = cvw[:, 0:4] + cvw[:, 4:8]
    bx2 = bx2_ref[0, 0]
    cxu = cv[:, 0:3] + bx2 * xsum_ref[0, :][None, :]
    csumu = cv[:, 3:4] + n * bx2

    maggp = jnp.sum(m.reshape(bi, nh, 2 * d), axis=1)
    maggu = maggp[:, :d] + maggp[:, d:]

    bj_diag = jnp.dot(hi, we1b_ref[:, :], preferred_element_type=f32)
    m0d = aip + bj_diag
    md = _silu(jnp.dot(_silu(m0d), we2_ref[:, :], preferred_element_type=f32)
               + be2_ref[0, :][None, :])
    td = _silu(jnp.dot(md, wx1_ref[:, :], preferred_element_type=f32)
               + bx1_ref[0, :][None, :])
    cd = jnp.sum(td * wx2r_ref[0, :][None, :], axis=1, keepdims=True) + bx2

    csum = csumu - cd
    cx = cxu - cd * xi
    ox_ref[:, :] = xi + (xi * csum - cx) * (1.0 / (n - 1))

    magg = maggu - md
    g = _silu(jnp.dot(hi, wh1a_ref[:, :], preferred_element_type=f32)
              + jnp.dot(magg, wh1b_ref[:, :], preferred_element_type=f32)
              + bh1_ref[0, :][None, :])
    hupd = jnp.dot(g, wh2_ref[:, :], preferred_element_type=f32) \
        + bh2_ref[0, :][None, :]
    oh_ref[:, :] = _silu(hi + hupd)


def _egnn_layer(h, x, We1, be1, We2, be2, Wx1, bx1, Wx2, bx2,
                Wh1, bh1, Wh2, bh2, *, bi, interpret=False):
    n, d = h.shape
    f32 = jnp.float32
    nh = n // 2
    xTa = x[:nh].T
    xTb = x[nh:].T
    one = jnp.ones((nh, 1), f32)
    xjf = jnp.tile(jnp.concatenate([x[:nh], one, x[nh:], one], axis=1),
                   (bi, 2 * d // 8))
    xsum = jnp.sum(x, axis=0, keepdims=True)

    we1a = We1[:d]
    we1b = We1[d:2 * d]
    we1c = We1[2 * d:2 * d + 1]
    z1 = jnp.zeros((1, d), f32)
    we1ca = jnp.concatenate([we1c, z1], axis=1)
    we1cb = jnp.concatenate([z1, we1c], axis=1)
    zd = jnp.zeros((d, d), f32)
    we2d = jnp.concatenate(
        [jnp.concatenate([We2, zd], axis=1),
         jnp.concatenate([zd, We2], axis=1)], axis=0)
    wx1d = jnp.concatenate(
        [jnp.concatenate([Wx1, zd], axis=1),
         jnp.concatenate([zd, Wx1], axis=1)], axis=0)
    be2d = jnp.tile(be2.reshape(1, d), (1, 2))
    bx1d = jnp.tile(bx1.reshape(1, d), (1, 2))
    zc = jnp.zeros((d, 1), f32)
    colA = jnp.concatenate([Wx2, zc], axis=0)
    colB = jnp.concatenate([zc, Wx2], axis=0)
    wx2w = jnp.tile(jnp.concatenate([colA] * 4 + [colB] * 4, axis=1),
                    (1, 2 * d // 8))
    wx2r = Wx2.T
    bx2m = bx2.reshape(1, 1)
    wh1a = Wh1[:d]
    wh1b = Wh1[d:]

    full = lambda shape: pl.BlockSpec(shape, lambda i: (0, 0))
    body = functools.partial(_layer_body, bi=bi, n=n, d=d)
    return pl.pallas_call(
        body,
        grid=(n // bi,),
        in_specs=[
            full((n, d)),
            pl.BlockSpec((bi, d), lambda i: (i, 0)),
            pl.BlockSpec((bi, 3), lambda i: (i, 0)),
            full((3, nh)),
            full((3, nh)),
            full((bi * nh, 2 * d)),
            full((d, d)),
            full((d, d)),
            full((1, 2 * d)),
            full((1, 2 * d)),
            full((1, d)),
            full((d, d)),
            full((2 * d, 2 * d)),
            full((1, d)),
            full((1, 2 * d)),
            full((d, d)),
            full((2 * d, 2 * d)),
            full((1, d)),
            full((1, 2 * d)),
            full((1, d)),
            full((2 * d, 2 * d)),
            full((1, 1)),
            full((1, 3)),
            full((d, d)),
            full((d, d)),
            full((1, d)),
            full((d, d)),
            full((1, d)),
        ],
        out_specs=[
            pl.BlockSpec((bi, d), lambda i: (i, 0)),
            pl.BlockSpec((bi, 3), lambda i: (i, 0)),
        ],
        out_shape=[
            jax.ShapeDtypeStruct((n, d), jnp.float32),
            jax.ShapeDtypeStruct((n, 3), jnp.float32),
        ],
        interpret=interpret,
    )(h, h, x, xTa, xTb, xjf, we1a, we1b, we1ca, we1cb, be1.reshape(1, d),
      We2, we2d, be2.reshape(1, d), be2d, Wx1, wx1d, bx1.reshape(1, d), bx1d,
      wx2r, wx2w, bx2m, xsum, wh1a, wh1b, bh1.reshape(1, d),
      Wh2, bh2.reshape(1, d))


def kernel(h, x, We1, be1, We2, be2, Wx1, bx1, Wx2, bx2, Wh1, bh1, Wh2, bh2):
    L = We1.shape[0]
    for l in range(L):
        h, x = _egnn_layer(h, x, We1[l], be1[l], We2[l], be2[l],
                           Wx1[l], bx1[l], Wx2[l], bx2[l],
                           Wh1[l], bh1[l], Wh2[l], bh2[l], bi=32)
    return (h, x)


# speedup vs baseline: 1.1683x; 1.0318x over previous
import functools

import jax
import jax.numpy as jnp
from jax.experimental import pallas as pl


def _silu(v):
    s = v * 0.5
    return s * jnp.tanh(s) + s


def _silu_pre(s):
    return s * jnp.tanh(s) + s


def _layer_body(h_ref, hi_ref, xi_ref, xTa_ref, xTb_ref, xjf_ref,
                we1a_ref, we1b_ref, we1ca_ref, we1cb_ref, be1_ref,
                we2_ref, we2d_ref, be2_ref, be2d_ref,
                wx1_ref, wx1d_ref, bx1_ref, bx1d_ref,
                wx2r_ref, wx2w_ref, bx2_ref, xsum_ref,
                wh1a_ref, wh1b_ref, bh1_ref, wh2_ref, bh2_ref,
                oh_ref, ox_ref, *, bi, n, d):
    f32 = jnp.float32
    nh = n // 2

    h_all = h_ref[:, :]
    hi = hi_ref[:, :]
    xi = xi_ref[:, :]

    d2a = jnp.zeros((bi, nh), f32)
    d2b = jnp.zeros((bi, nh), f32)
    for k in range(3):
        dka = xi[:, k:k + 1] - xTa_ref[k:k + 1, :]
        dkb = xi[:, k:k + 1] - xTb_ref[k:k + 1, :]
        d2a = d2a + dka * dka
        d2b = d2b + dkb * dkb

    ai = jnp.dot(hi, we1a_ref[:, :], preferred_element_type=f32)
    bj = jnp.dot(h_all, we1b_ref[:, :], preferred_element_type=f32)
    aip = ai + be1_ref[0, :][None, :]
    aip2 = jnp.concatenate([aip, aip], axis=1)
    bjp2 = jnp.concatenate([bj[:nh, :], bj[nh:, :]], axis=1)
    m0 = (aip2[:, None, :] + bjp2[None, :, :]
          + d2a[:, :, None] * we1ca_ref[0, :][None, None, :]
          + d2b[:, :, None] * we1cb_ref[0, :][None, None, :])
    m1 = _silu_pre(m0).reshape(bi * nh, 2 * d)
    m = _silu_pre(jnp.dot(m1, we2d_ref[:, :], preferred_element_type=f32)
                  + be2d_ref[0, :][None, :])

    t = _silu_pre(jnp.dot(m, wx1d_ref[:, :], preferred_element_type=f32)
                  + bx1d_ref[0, :][None, :])
    cc = jnp.dot(t, wx2w_ref[:, :], preferred_element_type=f32)
    u = cc * xjf_ref[:, :]
    cvw = jnp.sum(u.reshape(bi, nh, 2 * d), axis=1)
    cv = cvw[:, 0:4] + cvw[:, 4:8]
    bx2 = bx2_ref[0, 0]
    cxu = cv[:, 0:3] + bx2 * xsum_ref[0, :][None, :]
    csumu = cv[:, 3:4] + n * bx2

    maggp = jnp.sum(m.reshape(bi, nh, 2 * d), axis=1)
    maggu = maggp[:, :d] + maggp[:, d:]

    bj_diag = jnp.dot(hi, we1b_ref[:, :], preferred_element_type=f32)
    m0d = aip + bj_diag
    md = _silu_pre(jnp.dot(_silu_pre(m0d), we2_ref[:, :],
                          preferred_element_type=f32)
                  + be2_ref[0, :][None, :])
    td = _silu_pre(jnp.dot(md, wx1_ref[:, :], preferred_element_type=f32)
                  + bx1_ref[0, :][None, :])
    cd = jnp.sum(td * wx2r_ref[0, :][None, :], axis=1, keepdims=True) + bx2

    csum = csumu - cd
    cx = cxu - cd * xi
    ox_ref[:, :] = xi + (xi * csum - cx) * (1.0 / (n - 1))

    magg = maggu - md
    g = _silu_pre(jnp.dot(hi, wh1a_ref[:, :], preferred_element_type=f32)
                  + jnp.dot(magg, wh1b_ref[:, :], preferred_element_type=f32)
                  + bh1_ref[0, :][None, :])
    hupd = jnp.dot(g, wh2_ref[:, :], preferred_element_type=f32) \
        + bh2_ref[0, :][None, :]
    oh_ref[:, :] = _silu(hi + hupd)


def _egnn_layer(h, x, We1, be1, We2, be2, Wx1, bx1, Wx2, bx2,
                Wh1, bh1, Wh2, bh2, *, bi, interpret=False):
    n, d = h.shape
    f32 = jnp.float32
    nh = n // 2
    xTa = x[:nh].T
    xTb = x[nh:].T
    one = jnp.ones((nh, 1), f32)
    xjf = jnp.tile(jnp.concatenate([x[:nh], one, x[nh:], one], axis=1),
                   (bi, 2 * d // 8))
    xsum = jnp.sum(x, axis=0, keepdims=True)

    We2 = 0.5 * We2
    be2 = 0.5 * be2
    Wx1 = 0.5 * Wx1
    bx1 = 0.5 * bx1
    we1a = 0.5 * We1[:d]
    we1b = 0.5 * We1[d:2 * d]
    we1c = 0.5 * We1[2 * d:2 * d + 1]
    z1 = jnp.zeros((1, d), f32)
    we1ca = jnp.concatenate([we1c, z1], axis=1)
    we1cb = jnp.concatenate([z1, we1c], axis=1)
    zd = jnp.zeros((d, d), f32)
    we2d = jnp.concatenate(
        [jnp.concatenate([We2, zd], axis=1),
         jnp.concatenate([zd, We2], axis=1)], axis=0)
    wx1d = jnp.concatenate(
        [jnp.concatenate([Wx1, zd], axis=1),
         jnp.concatenate([zd, Wx1], axis=1)], axis=0)
    be2d = jnp.tile(be2.reshape(1, d), (1, 2))
    bx1d = jnp.tile(bx1.reshape(1, d), (1, 2))
    zc = jnp.zeros((d, 1), f32)
    colA = jnp.concatenate([Wx2, zc], axis=0)
    colB = jnp.concatenate([zc, Wx2], axis=0)
    wx2w = jnp.tile(jnp.concatenate([colA] * 4 + [colB] * 4, axis=1),
                    (1, 2 * d // 8))
    wx2r = Wx2.T
    bx2m = bx2.reshape(1, 1)
    wh1a = 0.5 * Wh1[:d]
    wh1b = 0.5 * Wh1[d:]
    bh1 = 0.5 * bh1

    full = lambda shape: pl.BlockSpec(shape, lambda i: (0, 0))
    body = functools.partial(_layer_body, bi=bi, n=n, d=d)
    return pl.pallas_call(
        body,
        grid=(n // bi,),
        in_specs=[
            full((n, d)),
            pl.BlockSpec((bi, d), lambda i: (i, 0)),
            pl.BlockSpec((bi, 3), lambda i: (i, 0)),
            full((3, nh)),
            full((3, nh)),
            full((bi * nh, 2 * d)),
            full((d, d)),
            full((d, d)),
            full((1, 2 * d)),
            full((1, 2 * d)),
            full((1, d)),
            full((d, d)),
            full((2 * d, 2 * d)),
            full((1, d)),
            full((1, 2 * d)),
            full((d, d)),
            full((2 * d, 2 * d)),
            full((1, d)),
            full((1, 2 * d)),
            full((1, d)),
            full((2 * d, 2 * d)),
            full((1, 1)),
            full((1, 3)),
            full((d, d)),
            full((d, d)),
            full((1, d)),
            full((d, d)),
            full((1, d)),
        ],
        out_specs=[
            pl.BlockSpec((bi, d), lambda i: (i, 0)),
            pl.BlockSpec((bi, 3), lambda i: (i, 0)),
        ],
        out_shape=[
            jax.ShapeDtypeStruct((n, d), jnp.float32),
            jax.ShapeDtypeStruct((n, 3), jnp.float32),
        ],
        interpret=interpret,
    )(h, h, x, xTa, xTb, xjf, we1a, we1b, we1ca, we1cb,
      0.5 * be1.reshape(1, d),
      We2, we2d, be2.reshape(1, d), be2d, Wx1, wx1d, bx1.reshape(1, d), bx1d,
      wx2r, wx2w, bx2m, xsum, wh1a, wh1b, bh1.reshape(1, d),
      Wh2, bh2.reshape(1, d))


def kernel(h, x, We1, be1, We2, be2, Wx1, bx1, Wx2, bx2, Wh1, bh1, Wh2, bh2):
    L = We1.shape[0]
    for l in range(L):
        h, x = _egnn_layer(h, x, We1[l], be1[l], We2[l], be2[l],
                           Wx1[l], bx1[l], Wx2[l], bx2[l],
                           Wh1[l], bh1[l], Wh2[l], bh2[l], bi=32)
    return (h, x)


# speedup vs baseline: 1.1724x; 1.0035x over previous
import functools

import jax
import jax.numpy as jnp
from jax.experimental import pallas as pl


def _silu(v):
    s = v * 0.5
    return s * jnp.tanh(s) + s


def _silu_pre(s):
    return s * jnp.tanh(s) + s


def _layer_body(h_ref, hi_ref, xi_ref, xTa_ref, xTb_ref, xjf_ref,
                we1a_ref, we1b_ref, we1cc_ref, be1_ref,
                we2_ref, we2d_ref, be2_ref, be2d_ref,
                wx1_ref, wx1d_ref, bx1_ref, bx1d_ref,
                wx2r_ref, wx2w_ref, bx2_ref, xsum_ref,
                wh1a_ref, wh1b_ref, bh1_ref, wh2_ref, bh2_ref,
                oh_ref, ox_ref, *, bi, n, d):
    f32 = jnp.float32
    nh = n // 2

    h_all = h_ref[:, :]
    hi = hi_ref[:, :]
    xi = xi_ref[:, :]

    d2a = jnp.zeros((bi, nh), f32)
    d2b = jnp.zeros((bi, nh), f32)
    for k in range(3):
        dka = xi[:, k:k + 1] - xTa_ref[k:k + 1, :]
        dkb = xi[:, k:k + 1] - xTb_ref[k:k + 1, :]
        d2a = d2a + dka * dka
        d2b = d2b + dkb * dkb

    ai = jnp.dot(hi, we1a_ref[:, :], preferred_element_type=f32)
    bj = jnp.dot(h_all, we1b_ref[:, :], preferred_element_type=f32)
    aip = ai + be1_ref[0, :][None, :]
    aip2 = jnp.concatenate([aip, aip], axis=1)
    bjp2 = jnp.concatenate([bj[:nh, :], bj[nh:, :]], axis=1)
    dcat = jnp.concatenate(
        [jnp.broadcast_to(d2a[:, :, None], (bi, nh, d)),
         jnp.broadcast_to(d2b[:, :, None], (bi, nh, d))], axis=2)
    m0 = (dcat * we1cc_ref[0, :][None, None, :]
          + (aip2[:, None, :] + bjp2[None, :, :]))
    m1 = _silu_pre(m0).reshape(bi * nh, 2 * d)
    m = _silu_pre(jnp.dot(m1, we2d_ref[:, :], preferred_element_type=f32)
                  + be2d_ref[0, :][None, :])

    t = _silu_pre(jnp.dot(m, wx1d_ref[:, :], preferred_element_type=f32)
                  + bx1d_ref[0, :][None, :])
    cc = jnp.dot(t, wx2w_ref[:, :], preferred_element_type=f32)
    u = cc * xjf_ref[:, :]
    cvw = jnp.sum(u.reshape(bi, nh, 2 * d), axis=1)
    cv = cvw[:, 0:4] + cvw[:, 4:8]
    bx2 = bx2_ref[0, 0]
    cxu = cv[:, 0:3] + bx2 * xsum_ref[0, :][None, :]
    csumu = cv[:, 3:4] + n * bx2

    maggp = jnp.sum(m.reshape(bi, nh, 2 * d), axis=1)
    maggu = maggp[:, :d] + maggp[:, d:]

    bj_diag = jnp.dot(hi, we1b_ref[:, :], preferred_element_type=f32)
    m0d = aip + bj_diag
    md = _silu_pre(jnp.dot(_silu_pre(m0d), we2_ref[:, :],
                          preferred_element_type=f32)
                  + be2_ref[0, :][None, :])
    td = _silu_pre(jnp.dot(md, wx1_ref[:, :], preferred_element_type=f32)
                  + bx1_ref[0, :][None, :])
    cd = jnp.sum(td * wx2r_ref[0, :][None, :], axis=1, keepdims=True) + bx2

    csum = csumu - cd
    cx = cxu - cd * xi
    ox_ref[:, :] = xi + (xi * csum - cx) * (1.0 / (n - 1))

    magg = maggu - md
    g = _silu_pre(jnp.dot(hi, wh1a_ref[:, :], preferred_element_type=f32)
                  + jnp.dot(magg, wh1b_ref[:, :], preferred_element_type=f32)
                  + bh1_ref[0, :][None, :])
    hupd = jnp.dot(g, wh2_ref[:, :], preferred_element_type=f32) \
        + bh2_ref[0, :][None, :]
    oh_ref[:, :] = _silu(hi + hupd)


def _egnn_layer(h, x, We1, be1, We2, be2, Wx1, bx1, Wx2, bx2,
                Wh1, bh1, Wh2, bh2, *, bi, interpret=False):
    n, d = h.shape
    f32 = jnp.float32
    nh = n // 2
    xTa = x[:nh].T
    xTb = x[nh:].T
    one = jnp.ones((nh, 1), f32)
    xjf = jnp.tile(jnp.concatenate([x[:nh], one, x[nh:], one], axis=1),
                   (bi, 2 * d // 8))
    xsum = jnp.sum(x, axis=0, keepdims=True)

    We2 = 0.5 * We2
    be2 = 0.5 * be2
    Wx1 = 0.5 * Wx1
    bx1 = 0.5 * bx1
    we1a = 0.5 * We1[:d]
    we1b = 0.5 * We1[d:2 * d]
    we1c = 0.5 * We1[2 * d:2 * d + 1]
    we1cc = jnp.concatenate([we1c, we1c], axis=1)
    zd = jnp.zeros((d, d), f32)
    we2d = jnp.concatenate(
        [jnp.concatenate([We2, zd], axis=1),
         jnp.concatenate([zd, We2], axis=1)], axis=0)
    wx1d = jnp.concatenate(
        [jnp.concatenate([Wx1, zd], axis=1),
         jnp.concatenate([zd, Wx1], axis=1)], axis=0)
    be2d = jnp.tile(be2.reshape(1, d), (1, 2))
    bx1d = jnp.tile(bx1.reshape(1, d), (1, 2))
    zc = jnp.zeros((d, 1), f32)
    colA = jnp.concatenate([Wx2, zc], axis=0)
    colB = jnp.concatenate([zc, Wx2], axis=0)
    wx2w = jnp.tile(jnp.concatenate([colA] * 4 + [colB] * 4, axis=1),
                    (1, 2 * d // 8))
    wx2r = Wx2.T
    bx2m = bx2.reshape(1, 1)
    wh1a = 0.5 * Wh1[:d]
    wh1b = 0.5 * Wh1[d:]
    bh1 = 0.5 * bh1

    full = lambda shape: pl.BlockSpec(shape, lambda i: (0, 0))
    body = functools.partial(_layer_body, bi=bi, n=n, d=d)
    return pl.pallas_call(
        body,
        grid=(n // bi,),
        in_specs=[
            full((n, d)),
            pl.BlockSpec((bi, d), lambda i: (i, 0)),
            pl.BlockSpec((bi, 3), lambda i: (i, 0)),
            full((3, nh)),
            full((3, nh)),
            full((bi * nh, 2 * d)),
            full((d, d)),
            full((d, d)),
            full((1, 2 * d)),
            full((1, d)),
            full((d, d)),
            full((2 * d, 2 * d)),
            full((1, d)),
            full((1, 2 * d)),
            full((d, d)),
            full((2 * d, 2 * d)),
            full((1, d)),
            full((1, 2 * d)),
            full((1, d)),
            full((2 * d, 2 * d)),
            full((1, 1)),
            full((1, 3)),
            full((d, d)),
            full((d, d)),
            full((1, d)),
            full((d, d)),
            full((1, d)),
        ],
        out_specs=[
            pl.BlockSpec((bi, d), lambda i: (i, 0)),
            pl.BlockSpec((bi, 3), lambda i: (i, 0)),
        ],
        out_shape=[
            jax.ShapeDtypeStruct((n, d), jnp.float32),
            jax.ShapeDtypeStruct((n, 3), jnp.float32),
        ],
        interpret=interpret,
    )(h, h, x, xTa, xTb, xjf, we1a, we1b, we1cc,
      0.5 * be1.reshape(1, d),
      We2, we2d, be2.reshape(1, d), be2d, Wx1, wx1d, bx1.reshape(1, d), bx1d,
      wx2r, wx2w, bx2m, xsum, wh1a, wh1b, bh1.reshape(1, d),
      Wh2, bh2.reshape(1, d))


def kernel(h, x, We1, be1, We2, be2, Wx1, bx1, Wx2, bx2, Wh1, bh1, Wh2, bh2):
    L = We1.shape[0]
    for l in range(L):
        h, x = _egnn_layer(h, x, We1[l], be1[l], We2[l], be2[l],
                           Wx1[l], bx1[l], Wx2[l], bx2[l],
                           Wh1[l], bh1[l], Wh2[l], bh2[l], bi=32)
    return (h, x)


# speedup vs baseline: 1.2876x; 1.0983x over previous
import functools

import jax
import jax.numpy as jnp
from jax.experimental import pallas as pl


def _silu(v):
    s = v * 0.5
    return s * jnp.tanh(s) + s


def _silu_pre(s):
    return s * jnp.tanh(s) + s


def _layer_body(h_ref, hi_ref, xi_ref, xTa_ref, xTb_ref, xjf_ref,
                we1a_ref, we1b_ref, we1cc_ref, be1_ref,
                we2_ref, we2d_ref, be2_ref, be2d_ref,
                wx1_ref, wx1d_ref, bx1_ref, bx1d_ref,
                wx2r_ref, wx2w_ref, bx2_ref, xsum_ref,
                wh1a_ref, wh1b_ref, bh1_ref, wh2_ref, bh2_ref,
                oh_ref, ox_ref, *, bi, n, d):
    f32 = jnp.float32
    nh = n // 2

    h_all = h_ref[:, :]
    hi = hi_ref[:, :]
    xi = xi_ref[:, :]

    d2a = jnp.zeros((bi, nh), f32)
    d2b = jnp.zeros((bi, nh), f32)
    for k in range(3):
        dka = xi[:, k:k + 1] - xTa_ref[k:k + 1, :]
        dkb = xi[:, k:k + 1] - xTb_ref[k:k + 1, :]
        d2a = d2a + dka * dka
        d2b = d2b + dkb * dkb

    ai = jnp.dot(hi, we1a_ref[:, :], preferred_element_type=f32)
    bj = jnp.dot(h_all, we1b_ref[:, :], preferred_element_type=f32)
    aip = ai + be1_ref[0, :][None, :]
    aip2 = jnp.concatenate([aip, aip], axis=1)
    bjp2 = jnp.concatenate([bj[:nh, :], bj[nh:, :]], axis=1)
    dcat = jnp.concatenate(
        [jnp.broadcast_to(d2a[:, :, None], (bi, nh, d)),
         jnp.broadcast_to(d2b[:, :, None], (bi, nh, d))], axis=2)
    m0 = (dcat * we1cc_ref[0, :][None, None, :]
          + (aip2[:, None, :] + bjp2[None, :, :]))
    m1 = _silu_pre(m0).reshape(bi * nh, 2 * d)
    m = _silu_pre(jnp.dot(m1, we2d_ref[:, :], preferred_element_type=f32)
                  + be2d_ref[0, :][None, :])

    t = _silu_pre(jnp.dot(m, wx1d_ref[:, :], preferred_element_type=f32)
                  + bx1d_ref[0, :][None, :])
    cc = jnp.dot(t, wx2w_ref[:, :], preferred_element_type=f32)
    u = cc.reshape(bi, nh, 2 * d) * xjf_ref[:, :][None, :, :]
    cvw = jnp.sum(u, axis=1)
    cv = cvw[:, 0:4] + cvw[:, 4:8]
    bx2 = bx2_ref[0, 0]
    cxu = cv[:, 0:3] + bx2 * xsum_ref[0, :][None, :]
    csumu = cv[:, 3:4] + n * bx2

    maggp = jnp.sum(m.reshape(bi, nh, 2 * d), axis=1)
    maggu = maggp[:, :d] + maggp[:, d:]

    bj_diag = jnp.dot(hi, we1b_ref[:, :], preferred_element_type=f32)
    m0d = aip + bj_diag
    md = _silu_pre(jnp.dot(_silu_pre(m0d), we2_ref[:, :],
                          preferred_element_type=f32)
                  + be2_ref[0, :][None, :])
    td = _silu_pre(jnp.dot(md, wx1_ref[:, :], preferred_element_type=f32)
                  + bx1_ref[0, :][None, :])
    cd = jnp.sum(td * wx2r_ref[0, :][None, :], axis=1, keepdims=True) + bx2

    csum = csumu - cd
    cx = cxu - cd * xi
    ox_ref[:, :] = xi + (xi * csum - cx) * (1.0 / (n - 1))

    magg = maggu - md
    g = _silu_pre(jnp.dot(hi, wh1a_ref[:, :], preferred_element_type=f32)
                  + jnp.dot(magg, wh1b_ref[:, :], preferred_element_type=f32)
                  + bh1_ref[0, :][None, :])
    hupd = jnp.dot(g, wh2_ref[:, :], preferred_element_type=f32) \
        + bh2_ref[0, :][None, :]
    oh_ref[:, :] = _silu(hi + hupd)


def _egnn_layer(h, x, We1, be1, We2, be2, Wx1, bx1, Wx2, bx2,
                Wh1, bh1, Wh2, bh2, *, bi, interpret=False):
    n, d = h.shape
    f32 = jnp.float32
    nh = n // 2
    xTa = x[:nh].T
    xTb = x[nh:].T
    one = jnp.ones((nh, 1), f32)
    xjf = jnp.tile(jnp.concatenate([x[:nh], one, x[nh:], one], axis=1),
                   (1, 2 * d // 8))
    xsum = jnp.sum(x, axis=0, keepdims=True)

    We2 = 0.5 * We2
    be2 = 0.5 * be2
    Wx1 = 0.5 * Wx1
    bx1 = 0.5 * bx1
    we1a = 0.5 * We1[:d]
    we1b = 0.5 * We1[d:2 * d]
    we1c = 0.5 * We1[2 * d:2 * d + 1]
    we1cc = jnp.concatenate([we1c, we1c], axis=1)
    zd = jnp.zeros((d, d), f32)
    we2d = jnp.concatenate(
        [jnp.concatenate([We2, zd], axis=1),
         jnp.concatenate([zd, We2], axis=1)], axis=0)
    wx1d = jnp.concatenate(
        [jnp.concatenate([Wx1, zd], axis=1),
         jnp.concatenate([zd, Wx1], axis=1)], axis=0)
    be2d = jnp.tile(be2.reshape(1, d), (1, 2))
    bx1d = jnp.tile(bx1.reshape(1, d), (1, 2))
    zc = jnp.zeros((d, 1), f32)
    colA = jnp.concatenate([Wx2, zc], axis=0)
    colB = jnp.concatenate([zc, Wx2], axis=0)
    wx2w = jnp.tile(jnp.concatenate([colA] * 4 + [colB] * 4, axis=1),
                    (1, 2 * d // 8))
    wx2r = Wx2.T
    bx2m = bx2.reshape(1, 1)
    wh1a = 0.5 * Wh1[:d]
    wh1b = 0.5 * Wh1[d:]
    bh1 = 0.5 * bh1

    full = lambda shape: pl.BlockSpec(shape, lambda i: (0, 0))
    body = functools.partial(_layer_body, bi=bi, n=n, d=d)
    return pl.pallas_call(
        body,
        grid=(n // bi,),
        in_specs=[
            full((n, d)),
            pl.BlockSpec((bi, d), lambda i: (i, 0)),
            pl.BlockSpec((bi, 3), lambda i: (i, 0)),
            full((3, nh)),
            full((3, nh)),
            full((nh, 2 * d)),
            full((d, d)),
            full((d, d)),
            full((1, 2 * d)),
            full((1, d)),
            full((d, d)),
            full((2 * d, 2 * d)),
            full((1, d)),
            full((1, 2 * d)),
            full((d, d)),
            full((2 * d, 2 * d)),
            full((1, d)),
            full((1, 2 * d)),
            full((1, d)),
            full((2 * d, 2 * d)),
            full((1, 1)),
            full((1, 3)),
            full((d, d)),
            full((d, d)),
            full((1, d)),
            full((d, d)),
            full((1, d)),
        ],
        out_specs=[
            pl.BlockSpec((bi, d), lambda i: (i, 0)),
            pl.BlockSpec((bi, 3), lambda i: (i, 0)),
        ],
        out_shape=[
            jax.ShapeDtypeStruct((n, d), jnp.float32),
            jax.ShapeDtypeStruct((n, 3), jnp.float32),
        ],
        interpret=interpret,
    )(h, h, x, xTa, xTb, xjf, we1a, we1b, we1cc,
      0.5 * be1.reshape(1, d),
      We2, we2d, be2.reshape(1, d), be2d, Wx1, wx1d, bx1.reshape(1, d), bx1d,
      wx2r, wx2w, bx2m, xsum, wh1a, wh1b, bh1.reshape(1, d),
      Wh2, bh2.reshape(1, d))


def kernel(h, x, We1, be1, We2, be2, Wx1, bx1, Wx2, bx2, Wh1, bh1, Wh2, bh2):
    L = We1.shape[0]
    for l in range(L):
        h, x = _egnn_layer(h, x, We1[l], be1[l], We2[l], be2[l],
                           Wx1[l], bx1[l], Wx2[l], bx2[l],
                           Wh1[l], bh1[l], Wh2[l], bh2[l], bi=32)
    return (h, x)
